# baseline (speedup 1.0000x reference)
import functools

import jax
import jax.numpy as jnp
from jax import lax
from jax.experimental import pallas as pl
from jax.experimental.pallas import tpu as pltpu
from jax.experimental.pallas import tpu_sc as plsc

N_NODES = 10000
N_EDGES = 320000
D = 128
NC = 2
NS = 16
EPT = N_EDGES // NS
EPW = N_EDGES // (NC * NS)
B = 80
NB = EPW // B
CH = 2000
BPC = CH // B
Z0 = 624
Z1 = N_NODES - (NS - 1) * Z0
EPS = 1e-10

_COS_COEF = (1.0, -0.5, 0.0416666641831398, -0.0013888863613829017,
             2.480055445630569e-05, -2.7534807145457307e-07,
             2.0603609929281674e-09, -9.722502317122128e-12)
_TWO_PI = 6.283185307179586
_INV_TWO_PI = 0.15915494309189535


def _cos_poly(theta):
    t = theta * jnp.float32(_INV_TWO_PI)
    k = (t + 0.5 * jnp.sign(t)).astype(jnp.int32).astype(jnp.float32)
    r = theta - k * jnp.float32(_TWO_PI)
    u = r * r
    acc = jnp.full_like(u, _COS_COEF[-1])
    for c in _COS_COEF[-2::-1]:
        acc = acc * u + jnp.float32(c)
    return acc


def _mm_body(x_ref, w_ref, b_ref, o_ref):
    acc = lax.dot_general(x_ref[...], w_ref[...],
                          (((1,), (1,)), ((), ())),
                          preferred_element_type=jnp.float32)
    o_ref[...] = (acc + b_ref[...]).astype(jnp.bfloat16)


def _matmul_bias_bf16(x, W, b):
    n = x.shape[0]
    blk = 2000
    return pl.pallas_call(
        _mm_body,
        grid=(n // blk,),
        in_specs=[
            pl.BlockSpec((blk, D), lambda i: (i, 0)),
            pl.BlockSpec((D, D), lambda i: (0, 0)),
            pl.BlockSpec((1, D), lambda i: (0, 0)),
        ],
        out_specs=pl.BlockSpec((blk, D), lambda i: (i, 0)),
        out_shape=jax.ShapeDtypeStruct((n, D), jnp.bfloat16),
    )(x, W, b.reshape(1, D))


_UNPACK_PERM = []
for _base in range(0, D, 32):
    for _k in range(16):
        _UNPACK_PERM.append(_base + _k)
        _UNPACK_PERM.append(_base + 16 + _k)


def _add_body(a_ref, b_ref, o_ref):
    o_ref[...] = a_ref[...] + b_ref[...]


def _final_add(parts):
    blk = 2000
    return pl.pallas_call(
        _add_body,
        grid=(N_NODES // blk,),
        in_specs=[
            pl.BlockSpec((blk, D), lambda i: (i, 0)),
            pl.BlockSpec((blk, D), lambda i: (N_NODES // blk + i, 0)),
        ],
        out_specs=pl.BlockSpec((blk, D), lambda i: (i, 0)),
        out_shape=jax.ShapeDtypeStruct((N_NODES, D), jnp.float32),
    )(parts, parts)


def _sc_body(h_hbm, src2_hbm, dst_hbm, v_hbm, zeros_hbm, zeros1d_hbm, out_hbm,
             h_v, sbig2, dbig, abuf0, abuf1, rsg0, rsg1, wbuf,
             rowsb0, rowsb1, rowsf0, rowsf1,
             red_all, red_stage,
             rs_sh, red_sh, out_sh,
             gsem0, gsem1, ssem0, ssem1, rsem0, rsem1):
    cid = lax.axis_index("c")
    sid = lax.axis_index("s")
    f32 = jnp.float32

    P2R = EPT // B
    tile_e0 = sid * EPT
    work_e0 = tile_e0 + cid * EPW

    pltpu.sync_copy(h_hbm, h_v)

    row0 = pl.multiple_of(sid * Z0, 8)
    @pl.when(sid < NS - 1)
    def _():
        pltpu.sync_copy(zeros_hbm.at[pl.ds(0, Z0)], out_sh.at[pl.ds(row0, Z0)])
    @pl.when(sid == NS - 1)
    def _():
        pltpu.sync_copy(zeros_hbm, out_sh.at[pl.ds((NS - 1) * Z0, Z1)])
    @pl.when(sid == 0)
    def _():
        pltpu.sync_copy(zeros1d_hbm, rs_sh)
    plsc.subcore_barrier()

    mx0 = jnp.full((16,), -jnp.inf, f32)
    mn0 = jnp.full((16,), jnp.inf, f32)

    def _m_chunk(c, carry):
        pltpu.sync_copy(src2_hbm.at[pl.ds(sid * P2R + c * BPC, BPC)], sbig2)

        def _m_k(k, carry):
            def _m_l(l, carry):
                mx, mn = carry
                s16 = sbig2[k, pl.ds(l * 16, 16)]
                hv = plsc.load_gather(h_v, [s16])
                return jnp.maximum(mx, hv), jnp.minimum(mn, hv)
            return lax.fori_loop(0, B // 16, _m_l, carry)
        return lax.fori_loop(0, BPC, _m_k, carry)

    mx, mn = lax.fori_loop(0, EPT // CH, _m_chunk, (mx0, mn0))
    red_stage[pl.ds(0, 16)] = mx
    red_stage[pl.ds(16, 16)] = mn
    pltpu.sync_copy(red_stage.at[pl.ds(0, 16)], red_sh.at[pl.ds(sid * 16, 16)])
    pltpu.sync_copy(red_stage.at[pl.ds(16, 16)],
                    red_sh.at[pl.ds(256 + sid * 16, 16)])
    plsc.subcore_barrier()
    pltpu.sync_copy(red_sh, red_all)

    def _red_body(i, carry):
        mx, mn = carry
        mx = jnp.maximum(mx, red_all[pl.ds(i * 16, 16)])
        mn = jnp.minimum(mn, red_all[pl.ds(256 + i * 16, 16)])
        return mx, mn
    mx, mn = lax.fori_loop(0, NS, _red_body, (mx0, mn0))
    m = (jnp.max(mx) - jnp.min(mn)) * f32(1.0)
    denom = jnp.full((16,), 2.0, f32) * m + f32(EPS)
    c_vec = jnp.full((16,), jnp.pi, f32) / denom

    def _att16(k, l):
        s16 = sbig2[k, pl.ds(l * 16, 16)]
        d16 = dbig[pl.ds(k * B + l * 16, 16)]
        hs = plsc.load_gather(h_v, [s16])
        hd = plsc.load_gather(h_v, [d16])
        return s16, _cos_poly(c_vec * (hs - hd))

    def _p2_wait_add(abuf, asem):
        pltpu.make_async_copy(abuf, rs_sh.at[sbig2.at[0]], asem).wait()

    def _p2_batch(k, abuf, asem, wait_prev):
        @pl.when(wait_prev)
        def _():
            _p2_wait_add(abuf, asem)

        def _a_body(l, _):
            _, att = _att16(k, l)
            abuf[pl.ds(l * 16, 16)] = att
            return 0
        lax.fori_loop(0, B // 16, _a_body, 0)
        pltpu.async_copy(abuf, rs_sh.at[sbig2.at[k]], asem, add=True)

    def _rs_chunk(c, _):
        pltpu.sync_copy(src2_hbm.at[pl.ds(sid * P2R + c * BPC, BPC)], sbig2)
        pltpu.sync_copy(dst_hbm.at[pl.ds(tile_e0 + c * CH, CH)], dbig)

        def _p2_pair(kk, _):
            k0 = 2 * kk
            nf = kk > 0
            _p2_batch(k0, abuf0, ssem0, nf)
            _p2_batch(k0 + 1, abuf1, ssem1, nf)
            return 0
        lax.fori_loop(0, BPC // 2, _p2_pair, 0)
        _p2_batch(BPC - 1, abuf0, ssem0, jnp.bool_(True))
        _p2_wait_add(abuf0, ssem0)
        _p2_wait_add(abuf1, ssem1)
        return 0
    lax.fori_loop(0, EPT // CH, _rs_chunk, 0)
    plsc.subcore_barrier()

    def _stage3(c):
        pltpu.sync_copy(
            src2_hbm.at[pl.ds(sid * P2R + cid * NB + c * BPC, BPC)], sbig2)
        pltpu.sync_copy(dst_hbm.at[pl.ds(work_e0 + c * CH, CH)], dbig)

    def _prefetch(k, rowsb, gsem, rsg, rsem):
        pltpu.async_copy(v_hbm.at[dbig.at[pl.ds(k * B, B)]], rowsb, gsem)
        pltpu.async_copy(rs_sh.at[sbig2.at[k]], rsg, rsem)

    def _batch(k, rowsb, gsem, rsg, rsem, ssem, rowsf):
        pltpu.make_async_copy(rs_sh.at[sbig2.at[0]], rsg, rsem).wait()

        def _w_body(l, _):
            _, att = _att16(k, l)
            r16 = rsg[pl.ds(l * 16, 16)]
            wbuf[pl.ds(l * 16, 16)] = att / r16 + f32(EPS)
            return 0
        lax.fori_loop(0, B // 16, _w_body, 0)

        pltpu.make_async_copy(
            v_hbm.at[dbig.at[pl.ds(k * B, B)]], rowsb, gsem).wait()

        mask_hi = jnp.full((16,), -65536, jnp.int32)
        shl16 = jnp.full((16,), 16, jnp.int32)
        def _scale_grp(g, _):
            w16 = wbuf[pl.ds(g * 16, 16)]
            for l in range(16):
                r = g * 16 + l
                w = w16[l]
                for ci in range(D // 32):
                    xi = rowsb[r, pl.ds(ci * 16, 16)]
                    lo = plsc.bitcast(lax.shift_left(xi, shl16), f32)
                    hi = plsc.bitcast(jnp.bitwise_and(xi, mask_hi), f32)
                    rowsf[r, pl.ds(ci * 32, 16)] = lo * w
                    rowsf[r, pl.ds(ci * 32 + 16, 16)] = hi * w
            return 0
        lax.fori_loop(0, B // 16, _scale_grp, 0)

        pltpu.async_copy(rowsf, out_sh.at[sbig2.at[k]], ssem, add=True)

    def _scat_wait(rowsf, ssem):
        pltpu.make_async_copy(rowsf, out_sh.at[sbig2.at[0]], ssem).wait()

    def _chunk3(c, _):
        _stage3(c)
        _prefetch(0, rowsb0, gsem0, rsg0, rsem0)
        _prefetch(1, rowsb1, gsem1, rsg1, rsem1)

        def _pair(kk, _):
            k0 = 2 * kk
            _batch(k0, rowsb0, gsem0, rsg0, rsem0, ssem0, rowsf0)
            _prefetch(k0 + 2, rowsb0, gsem0, rsg0, rsem0)
            _batch(k0 + 1, rowsb1, gsem1, rsg1, rsem1, ssem1, rowsf1)
            _scat_wait(rowsf0, ssem0)
            @pl.when(k0 + 3 < BPC)
            def _():
                _prefetch(k0 + 3, rowsb1, gsem1, rsg1, rsem1)
                _scat_wait(rowsf1, ssem1)
            return 0

        lax.fori_loop(0, BPC // 2, _pair, 0)
        _batch(BPC - 1, rowsb0, gsem0, rsg0, rsem0, ssem0, rowsf0)
        _scat_wait(rowsf0, ssem0)
        _scat_wait(rowsf1, ssem1)
        return 0

    lax.fori_loop(0, EPW // CH, _chunk3, 0)

    plsc.subcore_barrier()
    o0 = pl.multiple_of(cid * N_NODES + sid * Z0, 8)
    @pl.when(sid < NS - 1)
    def _():
        pltpu.sync_copy(out_sh.at[pl.ds(row0, Z0)], out_hbm.at[pl.ds(o0, Z0)])
    @pl.when(sid == NS - 1)
    def _():
        pltpu.sync_copy(out_sh.at[pl.ds((NS - 1) * Z0, Z1)],
                        out_hbm.at[pl.ds(cid * N_NODES + (NS - 1) * Z0, Z1)])


def _sc_edges(h, src2, dst, V, zeros, zeros1d):
    mesh = plsc.VectorSubcoreMesh(core_axis_name="c", subcore_axis_name="s")
    f32 = jnp.float32
    return pl.kernel(
        _sc_body,
        out_type=jax.ShapeDtypeStruct((NC * N_NODES, D), f32),
        mesh=mesh,
        scratch_types=[
            pltpu.VMEM((N_NODES,), f32),
            pltpu.VMEM((BPC, B), jnp.int32),
            pltpu.VMEM((CH,), jnp.int32),
            pltpu.VMEM((B,), f32),
            pltpu.VMEM((B,), f32),
            pltpu.VMEM((B,), f32),
            pltpu.VMEM((B,), f32),
            pltpu.VMEM((B,), f32),
            pltpu.VMEM((B, D // 2), jnp.int32),
            pltpu.VMEM((B, D // 2), jnp.int32),
            pltpu.VMEM((B, D), f32),
            pltpu.VMEM((B, D), f32),
            pltpu.VMEM((512,), f32),
            pltpu.VMEM((32,), f32),
            pltpu.VMEM_SHARED((N_NODES,), f32),
            pltpu.VMEM_SHARED((512,), f32),
            pltpu.VMEM_SHARED((N_NODES, D), f32),
            pltpu.SemaphoreType.DMA,
            pltpu.SemaphoreType.DMA,
            pltpu.SemaphoreType.DMA,
            pltpu.SemaphoreType.DMA,
            pltpu.SemaphoreType.DMA,
            pltpu.SemaphoreType.DMA,
        ],
        compiler_params=pltpu.CompilerParams(needs_layout_passes=False,
                                             use_tc_tiling_on_sc=False),
    )(h, src2, dst, V, zeros, zeros1d)


def kernel(x, edge_index, W1, b1, W2, b2):
    ei = edge_index.astype(jnp.int32)
    src, dst = ei[0], ei[1]

    Q = jax.nn.relu(x @ W2.T + b2)
    mu = jnp.mean(Q, axis=-1, keepdims=True)
    var = jnp.var(Q, axis=-1, keepdims=True)
    hn = (Q - mu) / jnp.sqrt(var + 1e-5)
    h = (jnp.sum(hn, axis=1)[:, None] ** 2)[:, 0]

    perm = jnp.asarray(_UNPACK_PERM, jnp.int32)
    Vb = _matmul_bias_bf16(x, W1[perm], b1[perm])
    Vp = lax.bitcast_convert_type(
        Vb.reshape(N_NODES, D // 2, 2), jnp.int32)
    src2 = src.reshape(N_EDGES // B, B)
    zeros = jnp.zeros((Z1, D), jnp.float32)
    zeros1d = jnp.zeros((N_NODES,), jnp.float32)
    parts = _sc_edges(h, src2, dst, Vp, zeros, zeros1d)
    return _final_add(parts)


# speedup vs baseline: 1.5392x; 1.5392x over previous
import functools

import jax
import jax.numpy as jnp
from jax import lax
from jax.experimental import pallas as pl
from jax.experimental.pallas import tpu as pltpu
from jax.experimental.pallas import tpu_sc as plsc

N_NODES = 10000
N_EDGES = 320000
D = 128
NC = 2
NS = 16
EPT = N_EDGES // NS
EPW = N_EDGES // (NC * NS)
B = 80
NB = EPW // B
CH = 2000
BPC = CH // B
Z0 = 624
Z1 = N_NODES - (NS - 1) * Z0
EPS = 1e-10

_COS_COEF = (1.0, -0.5, 0.0416666641831398, -0.0013888863613829017,
             2.480055445630569e-05, -2.7534807145457307e-07,
             2.0603609929281674e-09, -9.722502317122128e-12)
_TWO_PI = 6.283185307179586
_INV_TWO_PI = 0.15915494309189535


def _cos_poly(theta):
    t = theta * jnp.float32(_INV_TWO_PI)
    k = (t + 0.5 * jnp.sign(t)).astype(jnp.int32).astype(jnp.float32)
    r = theta - k * jnp.float32(_TWO_PI)
    u = r * r
    acc = jnp.full_like(u, _COS_COEF[-1])
    for c in _COS_COEF[-2::-1]:
        acc = acc * u + jnp.float32(c)
    return acc


def _mm_body(x_ref, w_ref, b_ref, o_ref):
    acc = lax.dot_general(x_ref[...], w_ref[...],
                          (((1,), (1,)), ((), ())),
                          preferred_element_type=jnp.float32)
    o_ref[...] = acc + b_ref[...]


def _matmul_bias(x, W, b):
    n = x.shape[0]
    blk = 2000
    return pl.pallas_call(
        _mm_body,
        grid=(n // blk,),
        in_specs=[
            pl.BlockSpec((blk, D), lambda i: (i, 0)),
            pl.BlockSpec((D, D), lambda i: (0, 0)),
            pl.BlockSpec((1, D), lambda i: (0, 0)),
        ],
        out_specs=pl.BlockSpec((blk, D), lambda i: (i, 0)),
        out_shape=jax.ShapeDtypeStruct((n, D), jnp.float32),
    )(x, W, b.reshape(1, D))


def _add_body(a_ref, b_ref, o_ref):
    o_ref[...] = a_ref[...] + b_ref[...]


def _final_add(parts):
    blk = 2000
    return pl.pallas_call(
        _add_body,
        grid=(N_NODES // blk,),
        in_specs=[
            pl.BlockSpec((blk, D), lambda i: (i, 0)),
            pl.BlockSpec((blk, D), lambda i: (N_NODES // blk + i, 0)),
        ],
        out_specs=pl.BlockSpec((blk, D), lambda i: (i, 0)),
        out_shape=jax.ShapeDtypeStruct((N_NODES, D), jnp.float32),
    )(parts, parts)


def _sc_body(h_hbm, src_hbm, dst_hbm, v_hbm, zeros_hbm, out_hbm,
             h_v, rs_v, sbig, dbig, sbuf0, sbuf1, wbuf, rows0, rows1,
             red_all, red_stage,
             rs_sh, red_sh, out_sh,
             gsem0, gsem1, ssem0, ssem1, tsem0, tsem1):
    cid = lax.axis_index("c")
    sid = lax.axis_index("s")
    f32 = jnp.float32

    tile_e0 = sid * EPT
    work_e0 = tile_e0 + cid * EPW

    with jax.named_scope("p0_stage"):
        pltpu.sync_copy(h_hbm, h_v)

        def _zero_rs(i, _):
            rs_v[pl.ds(i * 16, 16)] = jnp.zeros((16,), f32)
            return 0
        lax.fori_loop(0, N_NODES // 16, _zero_rs, 0)

        row0 = pl.multiple_of(sid * Z0, 8)
        @pl.when(sid < NS - 1)
        def _():
            pltpu.sync_copy(zeros_hbm.at[pl.ds(0, Z0)],
                            out_sh.at[pl.ds(row0, Z0)])
        @pl.when(sid == NS - 1)
        def _():
            pltpu.sync_copy(zeros_hbm, out_sh.at[pl.ds((NS - 1) * Z0, Z1)])
        @pl.when(sid == 0)
        def _():
            pltpu.sync_copy(rs_v, rs_sh)
        plsc.subcore_barrier()

    mx0 = jnp.full((16,), -jnp.inf, f32)
    mn0 = jnp.full((16,), jnp.inf, f32)

    def _m_chunk(c, carry):
        pltpu.sync_copy(src_hbm.at[pl.ds(tile_e0 + c * CH, CH)], sbig)

        def _m_body(i, carry):
            mx, mn = carry
            s16 = sbig[pl.ds(i * 16, 16)]
            hv = plsc.load_gather(h_v, [s16])
            return jnp.maximum(mx, hv), jnp.minimum(mn, hv)
        return lax.fori_loop(0, CH // 16, _m_body, carry)

    with jax.named_scope("p1_m"):
        mx, mn = lax.fori_loop(0, EPT // CH, _m_chunk, (mx0, mn0))
        red_stage[pl.ds(0, 16)] = mx
        red_stage[pl.ds(16, 16)] = mn
        pltpu.sync_copy(red_stage.at[pl.ds(0, 16)],
                        red_sh.at[pl.ds(sid * 16, 16)])
        pltpu.sync_copy(red_stage.at[pl.ds(16, 16)],
                        red_sh.at[pl.ds(256 + sid * 16, 16)])
        plsc.subcore_barrier()
        pltpu.sync_copy(red_sh, red_all)

        def _red_body(i, carry):
            mx, mn = carry
            mx = jnp.maximum(mx, red_all[pl.ds(i * 16, 16)])
            mn = jnp.minimum(mn, red_all[pl.ds(256 + i * 16, 16)])
            return mx, mn
        mx, mn = lax.fori_loop(0, NS, _red_body, (mx0, mn0))
        m = (jnp.max(mx) - jnp.min(mn)) * f32(1.0)
    denom = jnp.full((16,), 2.0, f32) * m + f32(EPS)
    c_vec = jnp.full((16,), jnp.pi, f32) / denom

    def _att16(off16):
        s16 = sbig[pl.ds(off16, 16)]
        d16 = dbig[pl.ds(off16, 16)]
        hs = plsc.load_gather(h_v, [s16])
        hd = plsc.load_gather(h_v, [d16])
        return s16, _cos_poly(c_vec * (hs - hd))

    def _rs_chunk(c, _):
        pltpu.sync_copy(src_hbm.at[pl.ds(tile_e0 + c * CH, CH)], sbig)
        pltpu.sync_copy(dst_hbm.at[pl.ds(tile_e0 + c * CH, CH)], dbig)

        def _rs_body(i, _):
            s16, att = _att16(i * 16)
            plsc.addupdate_scatter(rs_v, [s16], att)
            return 0
        lax.fori_loop(0, CH // 16, _rs_body, 0)
        return 0
    with jax.named_scope("p2_rs"):
        lax.fori_loop(0, EPT // CH, _rs_chunk, 0)

    def _iota_fill(k, sbuf):
        def _iw(l, _):
            sbuf[pl.ds(l * 16, 16)] = (lax.iota(jnp.int32, 16)
                                       + k * B + l * 16)
            return 0
        lax.fori_loop(0, B // 16, _iw, 0)

    def _comb_issue(k, sbuf, sem):
        _iota_fill(k, sbuf)
        pltpu.async_copy(rs_v.at[pl.ds(k * B, B)], rs_sh.at[sbuf], sem,
                         add=True)

    def _comb_wait(k, sbuf, sem):
        pltpu.make_async_copy(rs_v.at[pl.ds(k * B, B)], rs_sh.at[sbuf],
                              sem).wait()

    NKC = N_NODES // B
    with jax.named_scope("p2_comb"):
        _comb_issue(0, sbuf0, ssem0)
        _comb_issue(1, sbuf1, ssem1)

        def _comb_pair(kk, _):
            k0 = 2 * kk
            _comb_wait(k0 - 2, sbuf0, ssem0)
            _comb_issue(k0, sbuf0, ssem0)
            _comb_wait(k0 - 1, sbuf1, ssem1)
            _comb_issue(k0 + 1, sbuf1, ssem1)
            return 0
        lax.fori_loop(1, NKC // 2, _comb_pair, 0)
        _comb_wait(NKC - 3, sbuf0, ssem0)
        _comb_issue(NKC - 1, sbuf0, ssem0)
        _comb_wait(NKC - 1, sbuf0, ssem0)
        _comb_wait(NKC - 2, sbuf1, ssem1)
        plsc.subcore_barrier()
        pltpu.sync_copy(rs_sh, rs_v)

    def _stage3(c):
        pltpu.sync_copy(src_hbm.at[pl.ds(work_e0 + c * CH, CH)], sbig)
        pltpu.sync_copy(dst_hbm.at[pl.ds(work_e0 + c * CH, CH)], dbig)

    def _gather_rows(k, rows, sem):
        idx = dbig.at[pl.ds(k * B, B)]
        pltpu.async_copy(v_hbm.at[idx], rows, sem)

    def _stage_sbuf(c, k, sbuf, sem):
        pltpu.async_copy(src_hbm.at[pl.ds(work_e0 + c * CH + k * B, B)],
                         sbuf, sem)

    def _batch(c, k, rows, gsem, sbuf, tsem, ssem):
        def _w_body(l, _):
            s16, att = _att16(k * B + l * 16)
            rsg = plsc.load_gather(rs_v, [s16])
            wbuf[pl.ds(l * 16, 16)] = att / rsg + f32(EPS)
            return 0
        lax.fori_loop(0, B // 16, _w_body, 0)

        pltpu.make_async_copy(
            v_hbm.at[dbig.at[pl.ds(k * B, B)]], rows, gsem).wait()

        def _scale_grp(g, _):
            w16 = wbuf[pl.ds(g * 16, 16)]
            for l in range(16):
                r = g * 16 + l
                w = w16[l]
                for cc in range(D // 16):
                    sl = pl.ds(cc * 16, 16)
                    rows[r, sl] = rows[r, sl] * w
            return 0
        lax.fori_loop(0, B // 16, _scale_grp, 0)

        pltpu.make_async_copy(
            src_hbm.at[pl.ds(work_e0 + c * CH + k * B, B)], sbuf, tsem).wait()
        pltpu.async_copy(rows, out_sh.at[sbuf], ssem, add=True)

    def _scat_wait(rows, sbuf, ssem):
        pltpu.make_async_copy(rows, out_sh.at[sbuf], ssem).wait()

    def _chunk3(c, _):
        _stage3(c)
        _gather_rows(0, rows0, gsem0)
        _stage_sbuf(c, 0, sbuf0, tsem0)
        _gather_rows(1, rows1, gsem1)
        _stage_sbuf(c, 1, sbuf1, tsem1)

        def _pair(kk, _):
            k0 = 2 * kk
            _batch(c, k0, rows0, gsem0, sbuf0, tsem0, ssem0)
            _batch(c, k0 + 1, rows1, gsem1, sbuf1, tsem1, ssem1)
            _scat_wait(rows0, sbuf0, ssem0)
            _gather_rows(k0 + 2, rows0, gsem0)
            _stage_sbuf(c, k0 + 2, sbuf0, tsem0)
            @pl.when(k0 + 3 < BPC)
            def _():
                _scat_wait(rows1, sbuf1, ssem1)
                _gather_rows(k0 + 3, rows1, gsem1)
                _stage_sbuf(c, k0 + 3, sbuf1, tsem1)
            return 0

        lax.fori_loop(0, BPC // 2, _pair, 0)
        _batch(c, BPC - 1, rows0, gsem0, sbuf0, tsem0, ssem0)
        _scat_wait(rows0, sbuf0, ssem0)
        _scat_wait(rows1, sbuf1, ssem1)
        return 0

    with jax.named_scope("p3_msg"):
        lax.fori_loop(0, EPW // CH, _chunk3, 0)

    with jax.named_scope("p4_drain"):
        plsc.subcore_barrier()
        o0 = pl.multiple_of(cid * N_NODES + sid * Z0, 8)
        @pl.when(sid < NS - 1)
        def _():
            pltpu.sync_copy(out_sh.at[pl.ds(row0, Z0)],
                            out_hbm.at[pl.ds(o0, Z0)])
        @pl.when(sid == NS - 1)
        def _():
            pltpu.sync_copy(out_sh.at[pl.ds((NS - 1) * Z0, Z1)],
                            out_hbm.at[pl.ds(cid * N_NODES + (NS - 1) * Z0,
                                             Z1)])


def _sc_edges(h, src, dst, V, zeros):
    mesh = plsc.VectorSubcoreMesh(core_axis_name="c", subcore_axis_name="s")
    f32 = jnp.float32
    return pl.kernel(
        _sc_body,
        out_type=jax.ShapeDtypeStruct((NC * N_NODES, D), f32),
        mesh=mesh,
        scratch_types=[
            pltpu.VMEM((N_NODES,), f32),
            pltpu.VMEM((N_NODES,), f32),
            pltpu.VMEM((CH,), jnp.int32),
            pltpu.VMEM((CH,), jnp.int32),
            pltpu.VMEM((B,), jnp.int32),
            pltpu.VMEM((B,), jnp.int32),
            pltpu.VMEM((B,), f32),
            pltpu.VMEM((B, D), f32),
            pltpu.VMEM((B, D), f32),
            pltpu.VMEM((512,), f32),
            pltpu.VMEM((32,), f32),
            pltpu.VMEM_SHARED((N_NODES,), f32),
            pltpu.VMEM_SHARED((512,), f32),
            pltpu.VMEM_SHARED((N_NODES, D), f32),
            pltpu.SemaphoreType.DMA,
            pltpu.SemaphoreType.DMA,
            pltpu.SemaphoreType.DMA,
            pltpu.SemaphoreType.DMA,
            pltpu.SemaphoreType.DMA,
            pltpu.SemaphoreType.DMA,
        ],
        compiler_params=pltpu.CompilerParams(needs_layout_passes=False),
    )(h, src, dst, V, zeros)


def kernel(x, edge_index, W1, b1, W2, b2):
    ei = edge_index.astype(jnp.int32)
    src, dst = ei[0], ei[1]

    Q = jax.nn.relu(x @ W2.T + b2)
    mu = jnp.mean(Q, axis=-1, keepdims=True)
    var = jnp.var(Q, axis=-1, keepdims=True)
    hn = (Q - mu) / jnp.sqrt(var + 1e-5)
    h = (jnp.sum(hn, axis=1)[:, None] ** 2)[:, 0]

    V = _matmul_bias(x, W1, b1)
    zeros = jnp.zeros((Z1, D), jnp.float32)
    parts = _sc_edges(h, src, dst, V, zeros)
    return _final_add(parts)


# speedup vs baseline: 1.7862x; 1.1604x over previous
import functools

import jax
import jax.numpy as jnp
from jax import lax
from jax.experimental import pallas as pl
from jax.experimental.pallas import tpu as pltpu
from jax.experimental.pallas import tpu_sc as plsc

N_NODES = 10000
N_EDGES = 320000
D = 128
NC = 2
NS = 16
EPT = N_EDGES // NS
EPW = N_EDGES // (NC * NS)
B = 80
NB = EPW // B
CH = 2000
BPC = CH // B
Z0 = 624
Z1 = N_NODES - (NS - 1) * Z0
EPS = 1e-10

_COS_COEF = (1.0, -0.5, 0.0416666641831398, -0.0013888863613829017,
             2.480055445630569e-05, -2.7534807145457307e-07,
             2.0603609929281674e-09, -9.722502317122128e-12)
_TWO_PI = 6.283185307179586
_INV_TWO_PI = 0.15915494309189535


def _cos_poly(theta):
    t = theta * jnp.float32(_INV_TWO_PI)
    k = (t + 0.5 * jnp.sign(t)).astype(jnp.int32).astype(jnp.float32)
    r = theta - k * jnp.float32(_TWO_PI)
    u = r * r
    acc = jnp.full_like(u, _COS_COEF[-1])
    for c in _COS_COEF[-2::-1]:
        acc = acc * u + jnp.float32(c)
    return acc


def _mm_body(x_ref, w_ref, b_ref, o_ref):
    acc = lax.dot_general(x_ref[...], w_ref[...],
                          (((1,), (1,)), ((), ())),
                          preferred_element_type=jnp.float32)
    o_ref[...] = acc + b_ref[...]


def _matmul_bias(x, W, b):
    n = x.shape[0]
    blk = 2000
    return pl.pallas_call(
        _mm_body,
        grid=(n // blk,),
        in_specs=[
            pl.BlockSpec((blk, D), lambda i: (i, 0)),
            pl.BlockSpec((D, D), lambda i: (0, 0)),
            pl.BlockSpec((1, D), lambda i: (0, 0)),
        ],
        out_specs=pl.BlockSpec((blk, D), lambda i: (i, 0)),
        out_shape=jax.ShapeDtypeStruct((n, D), jnp.float32),
    )(x, W, b.reshape(1, D))


def _add_body(a_ref, b_ref, o_ref):
    o_ref[...] = a_ref[...] + b_ref[...]


def _final_add(parts):
    blk = 2000
    return pl.pallas_call(
        _add_body,
        grid=(N_NODES // blk,),
        in_specs=[
            pl.BlockSpec((blk, D), lambda i: (i, 0)),
            pl.BlockSpec((blk, D), lambda i: (N_NODES // blk + i, 0)),
        ],
        out_specs=pl.BlockSpec((blk, D), lambda i: (i, 0)),
        out_shape=jax.ShapeDtypeStruct((N_NODES, D), jnp.float32),
    )(parts, parts)


def _sc_body(h_hbm, src_hbm, dst_hbm, v_hbm, zeros_hbm, out_hbm,
             h_v, rs_v, sbig, dbig, sbuf0, sbuf1, wbuf, rows0, rows1,
             red_all, red_stage,
             rs_sh, red_sh, out_sh,
             gsem0, gsem1, ssem0, ssem1, tsem0, tsem1):
    cid = lax.axis_index("c")
    sid = lax.axis_index("s")
    f32 = jnp.float32

    tile_e0 = sid * EPT
    work_e0 = tile_e0 + cid * EPW

    with jax.named_scope("p0_stage"):
        pltpu.sync_copy(h_hbm, h_v)

        def _zero_rs(i, _):
            rs_v[pl.ds(i * 16, 16)] = jnp.zeros((16,), f32)
            return 0
        lax.fori_loop(0, N_NODES // 16, _zero_rs, 0)

        row0 = pl.multiple_of(sid * Z0, 8)
        @pl.when(sid < NS - 1)
        def _():
            pltpu.sync_copy(zeros_hbm.at[pl.ds(0, Z0)],
                            out_sh.at[pl.ds(row0, Z0)])
        @pl.when(sid == NS - 1)
        def _():
            pltpu.sync_copy(zeros_hbm, out_sh.at[pl.ds((NS - 1) * Z0, Z1)])
        @pl.when(sid == 0)
        def _():
            pltpu.sync_copy(rs_v, rs_sh)
        plsc.subcore_barrier()

    mx0 = jnp.full((16,), -jnp.inf, f32)
    mn0 = jnp.full((16,), jnp.inf, f32)

    def _m_chunk(c, carry):
        pltpu.sync_copy(src_hbm.at[pl.ds(tile_e0 + c * CH, CH)], sbig)

        def _m_body(i, carry):
            mx, mn = carry
            s16 = sbig[pl.ds(i * 16, 16)]
            hv = plsc.load_gather(h_v, [s16])
            return jnp.maximum(mx, hv), jnp.minimum(mn, hv)
        return plsc.parallel_loop(0, CH // 16, unroll=4,
                                  carry=carry)(_m_body)

    with jax.named_scope("p1_m"):
        mx, mn = lax.fori_loop(0, EPT // CH, _m_chunk, (mx0, mn0))
        red_stage[pl.ds(0, 16)] = mx
        red_stage[pl.ds(16, 16)] = mn
        pltpu.sync_copy(red_stage.at[pl.ds(0, 16)],
                        red_sh.at[pl.ds(sid * 16, 16)])
        pltpu.sync_copy(red_stage.at[pl.ds(16, 16)],
                        red_sh.at[pl.ds(256 + sid * 16, 16)])
        plsc.subcore_barrier()
        pltpu.sync_copy(red_sh, red_all)

        def _red_body(i, carry):
            mx, mn = carry
            mx = jnp.maximum(mx, red_all[pl.ds(i * 16, 16)])
            mn = jnp.minimum(mn, red_all[pl.ds(256 + i * 16, 16)])
            return mx, mn
        mx, mn = lax.fori_loop(0, NS, _red_body, (mx0, mn0))
        m = (jnp.max(mx) - jnp.min(mn)) * f32(1.0)
    denom = jnp.full((16,), 2.0, f32) * m + f32(EPS)
    c_vec = jnp.full((16,), jnp.pi, f32) / denom

    def _att16(off16):
        s16 = sbig[pl.ds(off16, 16)]
        d16 = dbig[pl.ds(off16, 16)]
        hs = plsc.load_gather(h_v, [s16])
        hd = plsc.load_gather(h_v, [d16])
        return s16, _cos_poly(c_vec * (hs - hd))

    def _rs_chunk(c, _):
        pltpu.sync_copy(src_hbm.at[pl.ds(tile_e0 + c * CH, CH)], sbig)
        pltpu.sync_copy(dst_hbm.at[pl.ds(tile_e0 + c * CH, CH)], dbig)

        @plsc.parallel_loop(0, CH // 16, unroll=4)
        def _rs_body(i):
            s16, att = _att16(i * 16)
            plsc.addupdate_scatter(rs_v, [s16], att)
        return 0
    with jax.named_scope("p2_rs"):
        lax.fori_loop(0, EPT // CH, _rs_chunk, 0)

    def _iota_fill(k, sbuf):
        def _iw(l, _):
            sbuf[pl.ds(l * 16, 16)] = (lax.iota(jnp.int32, 16)
                                       + k * B + l * 16)
            return 0
        lax.fori_loop(0, B // 16, _iw, 0)

    def _comb_issue(k, sbuf, sem):
        _iota_fill(k, sbuf)
        pltpu.async_copy(rs_v.at[pl.ds(k * B, B)], rs_sh.at[sbuf], sem,
                         add=True)

    def _comb_wait(k, sbuf, sem):
        pltpu.make_async_copy(rs_v.at[pl.ds(k * B, B)], rs_sh.at[sbuf],
                              sem).wait()

    NKC = N_NODES // B
    with jax.named_scope("p2_comb"):
        _comb_issue(0, sbuf0, ssem0)
        _comb_issue(1, sbuf1, ssem1)

        def _comb_pair(kk, _):
            k0 = 2 * kk
            _comb_wait(k0 - 2, sbuf0, ssem0)
            _comb_issue(k0, sbuf0, ssem0)
            _comb_wait(k0 - 1, sbuf1, ssem1)
            _comb_issue(k0 + 1, sbuf1, ssem1)
            return 0
        lax.fori_loop(1, NKC // 2, _comb_pair, 0)
        _comb_wait(NKC - 3, sbuf0, ssem0)
        _comb_issue(NKC - 1, sbuf0, ssem0)
        _comb_wait(NKC - 1, sbuf0, ssem0)
        _comb_wait(NKC - 2, sbuf1, ssem1)
        plsc.subcore_barrier()
        pltpu.sync_copy(rs_sh, rs_v)

    def _stage3(c):
        pltpu.sync_copy(src_hbm.at[pl.ds(work_e0 + c * CH, CH)], sbig)
        pltpu.sync_copy(dst_hbm.at[pl.ds(work_e0 + c * CH, CH)], dbig)

    def _gather_rows(k, rows, sem):
        idx = dbig.at[pl.ds(k * B, B)]
        pltpu.async_copy(v_hbm.at[idx], rows, sem)

    def _stage_sbuf(c, k, sbuf, sem):
        pltpu.async_copy(src_hbm.at[pl.ds(work_e0 + c * CH + k * B, B)],
                         sbuf, sem)

    def _batch(c, k, rows, gsem, sbuf, tsem, ssem):
        @plsc.parallel_loop(0, B // 16, unroll=5)
        def _w_body(l):
            s16, att = _att16(k * B + l * 16)
            rsg = plsc.load_gather(rs_v, [s16])
            wbuf[pl.ds(l * 16, 16)] = att / rsg + f32(EPS)

        pltpu.make_async_copy(
            v_hbm.at[dbig.at[pl.ds(k * B, B)]], rows, gsem).wait()

        @plsc.parallel_loop(0, B // 16)
        def _scale_grp(g):
            w16 = wbuf[pl.ds(g * 16, 16)]
            for l in range(16):
                r = g * 16 + l
                w = w16[l]
                for cc in range(D // 16):
                    sl = pl.ds(cc * 16, 16)
                    rows[r, sl] = rows[r, sl] * w

        pltpu.make_async_copy(
            src_hbm.at[pl.ds(work_e0 + c * CH + k * B, B)], sbuf, tsem).wait()
        pltpu.async_copy(rows, out_sh.at[sbuf], ssem, add=True)

    def _scat_wait(rows, sbuf, ssem):
        pltpu.make_async_copy(rows, out_sh.at[sbuf], ssem).wait()

    def _chunk3(c, _):
        _stage3(c)
        _gather_rows(0, rows0, gsem0)
        _stage_sbuf(c, 0, sbuf0, tsem0)
        _gather_rows(1, rows1, gsem1)
        _stage_sbuf(c, 1, sbuf1, tsem1)

        def _pair(kk, _):
            k0 = 2 * kk
            _batch(c, k0, rows0, gsem0, sbuf0, tsem0, ssem0)
            _batch(c, k0 + 1, rows1, gsem1, sbuf1, tsem1, ssem1)
            _scat_wait(rows0, sbuf0, ssem0)
            _gather_rows(k0 + 2, rows0, gsem0)
            _stage_sbuf(c, k0 + 2, sbuf0, tsem0)
            @pl.when(k0 + 3 < BPC)
            def _():
                _scat_wait(rows1, sbuf1, ssem1)
                _gather_rows(k0 + 3, rows1, gsem1)
                _stage_sbuf(c, k0 + 3, sbuf1, tsem1)
            return 0

        lax.fori_loop(0, BPC // 2, _pair, 0)
        _batch(c, BPC - 1, rows0, gsem0, sbuf0, tsem0, ssem0)
        _scat_wait(rows0, sbuf0, ssem0)
        _scat_wait(rows1, sbuf1, ssem1)
        return 0

    with jax.named_scope("p3_msg"):
        lax.fori_loop(0, EPW // CH, _chunk3, 0)

    with jax.named_scope("p4_drain"):
        plsc.subcore_barrier()
        o0 = pl.multiple_of(cid * N_NODES + sid * Z0, 8)
        @pl.when(sid < NS - 1)
        def _():
            pltpu.sync_copy(out_sh.at[pl.ds(row0, Z0)],
                            out_hbm.at[pl.ds(o0, Z0)])
        @pl.when(sid == NS - 1)
        def _():
            pltpu.sync_copy(out_sh.at[pl.ds((NS - 1) * Z0, Z1)],
                            out_hbm.at[pl.ds(cid * N_NODES + (NS - 1) * Z0,
                                             Z1)])


def _sc_edges(h, src, dst, V, zeros):
    mesh = plsc.VectorSubcoreMesh(core_axis_name="c", subcore_axis_name="s")
    f32 = jnp.float32
    return pl.kernel(
        _sc_body,
        out_type=jax.ShapeDtypeStruct((NC * N_NODES, D), f32),
        mesh=mesh,
        scratch_types=[
            pltpu.VMEM((N_NODES,), f32),
            pltpu.VMEM((N_NODES,), f32),
            pltpu.VMEM((CH,), jnp.int32),
            pltpu.VMEM((CH,), jnp.int32),
            pltpu.VMEM((B,), jnp.int32),
            pltpu.VMEM((B,), jnp.int32),
            pltpu.VMEM((B,), f32),
            pltpu.VMEM((B, D), f32),
            pltpu.VMEM((B, D), f32),
            pltpu.VMEM((512,), f32),
            pltpu.VMEM((32,), f32),
            pltpu.VMEM_SHARED((N_NODES,), f32),
            pltpu.VMEM_SHARED((512,), f32),
            pltpu.VMEM_SHARED((N_NODES, D), f32),
            pltpu.SemaphoreType.DMA,
            pltpu.SemaphoreType.DMA,
            pltpu.SemaphoreType.DMA,
            pltpu.SemaphoreType.DMA,
            pltpu.SemaphoreType.DMA,
            pltpu.SemaphoreType.DMA,
        ],
        compiler_params=pltpu.CompilerParams(needs_layout_passes=False),
    )(h, src, dst, V, zeros)


def kernel(x, edge_index, W1, b1, W2, b2):
    ei = edge_index.astype(jnp.int32)
    src, dst = ei[0], ei[1]

    Q = jax.nn.relu(x @ W2.T + b2)
    mu = jnp.mean(Q, axis=-1, keepdims=True)
    var = jnp.var(Q, axis=-1, keepdims=True)
    hn = (Q - mu) / jnp.sqrt(var + 1e-5)
    h = (jnp.sum(hn, axis=1)[:, None] ** 2)[:, 0]

    V = _matmul_bias(x, W1, b1)
    zeros = jnp.zeros((Z1, D), jnp.float32)
    parts = _sc_edges(h, src, dst, V, zeros)
    return _final_add(parts)
